# baseline (speedup 1.0000x reference)
import functools
import math

import jax
import jax.numpy as jnp
from jax import lax
from jax.experimental import pallas as pl
from jax.experimental.pallas import tpu as pltpu
from jax.experimental.pallas import tpu_sc as plsc

D_MODEL = 128
SCALE = math.sqrt(D_MODEL)
NUM_CORES = 2
NUM_SUBCORES = 16
NUM_WORKERS = NUM_CORES * NUM_SUBCORES
CHUNK = 128
LANES = 16
NB = 5


def _make_kernel(batch: int):
    assert batch % (NUM_WORKERS * CHUNK * NB) == 0
    b_per_w = batch // NUM_WORKERS
    n_chunks = b_per_w // CHUNK
    n_groups = n_chunks // NB

    mesh = plsc.VectorSubcoreMesh(
        core_axis_name="c", subcore_axis_name="s",
        num_cores=NUM_CORES, num_subcores=NUM_SUBCORES)

    @functools.partial(
        pl.kernel,
        out_type=jax.ShapeDtypeStruct((batch, D_MODEL), jnp.float32),
        mesh=mesh,
        scratch_types=[
            pltpu.VMEM((n_chunks, CHUNK), jnp.int32),
            *[pltpu.VMEM((CHUNK, D_MODEL), jnp.float32) for _ in range(NB)],
            *[pltpu.SemaphoreType.DMA for _ in range(NB)],
        ],
    )
    def emb_kernel(idx_hbm, table_hbm, out_hbm, idx_v,
                   g0, g1, g2, g3, g4, gsem0, gsem1, gsem2, gsem3, gsem4):
        gbuf = (g0, g1, g2, g3, g4)
        gsem = (gsem0, gsem1, gsem2, gsem3, gsem4)
        wid = lax.axis_index("s") * NUM_CORES + lax.axis_index("c")
        base = wid * b_per_w
        pltpu.sync_copy(idx_hbm.at[wid], idx_v)

        def fire_gather(b, c):
            pltpu.async_copy(table_hbm.at[idx_v.at[c]], gbuf[b], gsem[b])

        def wait_gather(b):
            pltpu.make_async_copy(
                table_hbm.at[pl.ds(0, CHUNK)], gbuf[b], gsem[b]).wait()

        def scale_and_store(b, c):
            def row_body(i, carry):
                for j in range(D_MODEL // LANES):
                    sl = pl.ds(j * LANES, LANES)
                    gbuf[b][i, sl] = gbuf[b][i, sl] * SCALE
                return carry
            lax.fori_loop(0, CHUNK, row_body, 0, unroll=2)
            pltpu.sync_copy(
                gbuf[b], out_hbm.at[pl.ds(base + c * CHUNK, CHUNK)])

        for b in range(NB - 1):
            fire_gather(b, b)

        def group(gi, carry):
            for b in range(NB):
                c = gi * NB + b
                wait_gather(b)
                fire_gather((b + NB - 1) % NB, c + NB - 1)
                scale_and_store(b, c)
            return carry

        lax.fori_loop(0, n_groups - 1, group, 0)
        for b in range(NB):
            c = n_chunks - NB + b
            wait_gather(b)
            if b == 0:
                fire_gather(NB - 1, n_chunks - 1)
            scale_and_store(b, c)

    return emb_kernel


def kernel(x, table):
    batch = x.shape[0] * x.shape[1]
    idx = x.reshape(NUM_WORKERS, batch // (NUM_WORKERS * CHUNK), CHUNK)
    idx = idx.astype(jnp.int32)
    out = _make_kernel(batch)(idx, table)
    return out.reshape(x.shape[0], x.shape[1], D_MODEL)


# speedup vs baseline: 1.0174x; 1.0174x over previous
import functools
import math

import jax
import jax.numpy as jnp
from jax import lax
from jax.experimental import pallas as pl
from jax.experimental.pallas import tpu as pltpu
from jax.experimental.pallas import tpu_sc as plsc

D_MODEL = 128
SCALE = math.sqrt(D_MODEL)
NUM_CORES = 2
NUM_SUBCORES = 16
NUM_WORKERS = NUM_CORES * NUM_SUBCORES
CHUNK = 128
LANES = 16
NB = 5


def _make_kernel(batch: int):
    assert batch % (NUM_WORKERS * CHUNK * NB) == 0
    b_per_w = batch // NUM_WORKERS
    n_chunks = b_per_w // CHUNK
    n_groups = n_chunks // NB

    mesh = plsc.VectorSubcoreMesh(
        core_axis_name="c", subcore_axis_name="s",
        num_cores=NUM_CORES, num_subcores=NUM_SUBCORES)

    @functools.partial(
        pl.kernel,
        out_type=jax.ShapeDtypeStruct((batch, D_MODEL), jnp.float32),
        mesh=mesh,
        scratch_types=[
            pltpu.VMEM((n_chunks, CHUNK), jnp.int32),
            *[pltpu.VMEM((CHUNK, D_MODEL), jnp.float32) for _ in range(NB)],
            *[pltpu.SemaphoreType.DMA for _ in range(2 * NB)],
        ],
    )
    def emb_kernel(idx_hbm, table_hbm, out_hbm, idx_v,
                   g0, g1, g2, g3, g4,
                   gsem0, gsem1, gsem2, gsem3, gsem4,
                   ssem0, ssem1, ssem2, ssem3, ssem4):
        gbuf = (g0, g1, g2, g3, g4)
        gsem = (gsem0, gsem1, gsem2, gsem3, gsem4)
        ssem = (ssem0, ssem1, ssem2, ssem3, ssem4)
        wid = lax.axis_index("s") * NUM_CORES + lax.axis_index("c")
        base = wid * b_per_w
        pltpu.sync_copy(idx_hbm.at[wid], idx_v)

        def fire_gather(b, c):
            pltpu.async_copy(table_hbm.at[idx_v.at[c]], gbuf[b], gsem[b])

        def wait_gather(b):
            pltpu.make_async_copy(
                table_hbm.at[pl.ds(0, CHUNK)], gbuf[b], gsem[b]).wait()

        def fire_store(b, c):
            pltpu.async_copy(
                gbuf[b], out_hbm.at[pl.ds(base + c * CHUNK, CHUNK)], ssem[b])

        def wait_store(b):
            pltpu.make_async_copy(
                gbuf[b], out_hbm.at[pl.ds(0, CHUNK)], ssem[b]).wait()

        def scale(b):
            def row_body(i, carry):
                for j in range(D_MODEL // LANES):
                    sl = pl.ds(j * LANES, LANES)
                    gbuf[b][i, sl] = gbuf[b][i, sl] * SCALE
                return carry
            lax.fori_loop(0, CHUNK, row_body, 0, unroll=2)

        AHEAD = 3
        for b in range(AHEAD):
            fire_gather(b, b)
        for b in range(NB):
            c = b
            g = (b + AHEAD) % NB
            if b >= NB - AHEAD:
                wait_store(g)
            fire_gather(g, c + AHEAD)
            wait_gather(b)
            scale(b)
            fire_store(b, c)

        def group(gi, carry):
            for b in range(NB):
                c = gi * NB + b
                g = (b + AHEAD) % NB
                wait_store(g)
                fire_gather(g, c + AHEAD)
                wait_gather(b)
                scale(b)
                fire_store(b, c)
            return carry

        lax.fori_loop(1, n_groups - 1, group, 0)
        for b in range(NB):
            c = n_chunks - NB + b
            g = (b + AHEAD) % NB
            if c + AHEAD < n_chunks:
                wait_store(g)
                fire_gather(g, c + AHEAD)
            wait_gather(b)
            scale(b)
            fire_store(b, c)
        for b in range(NB):
            wait_store(b)

    return emb_kernel


def kernel(x, table):
    batch = x.shape[0] * x.shape[1]
    idx = x.reshape(NUM_WORKERS, batch // (NUM_WORKERS * CHUNK), CHUNK)
    idx = idx.astype(jnp.int32)
    out = _make_kernel(batch)(idx, table)
    return out.reshape(x.shape[0], x.shape[1], D_MODEL)


# speedup vs baseline: 1.0534x; 1.0353x over previous
import functools
import math

import jax
import jax.numpy as jnp
from jax import lax
from jax.experimental import pallas as pl
from jax.experimental.pallas import tpu as pltpu
from jax.experimental.pallas import tpu_sc as plsc

D_MODEL = 128
SCALE = math.sqrt(D_MODEL)
NUM_CORES = 2
NUM_SUBCORES = 16
NUM_WORKERS = NUM_CORES * NUM_SUBCORES
CHUNK = 128
HALF = CHUNK // 2
LANES = 16
NB = 4
NS = 4
AHEAD = 3


def _make_kernel(batch: int):
    assert batch % (NUM_WORKERS * CHUNK * NB) == 0
    b_per_w = batch // NUM_WORKERS
    n_chunks = b_per_w // CHUNK
    n_groups = n_chunks // NB

    mesh = plsc.VectorSubcoreMesh(
        core_axis_name="c", subcore_axis_name="s",
        num_cores=NUM_CORES, num_subcores=NUM_SUBCORES)

    @functools.partial(
        pl.kernel,
        out_type=jax.ShapeDtypeStruct((batch, D_MODEL), jnp.float32),
        mesh=mesh,
        scratch_types=[
            pltpu.VMEM((n_chunks, CHUNK), jnp.int32),
            *[pltpu.VMEM((CHUNK, D_MODEL), jnp.float32) for _ in range(NB)],
            pltpu.VMEM_SHARED(
                (NUM_SUBCORES, NS, HALF, D_MODEL), jnp.float32),
            *[pltpu.SemaphoreType.DMA for _ in range(NB + 2 * NS)],
        ],
    )
    def emb_kernel(idx_hbm, table_hbm, out_hbm, idx_v,
                   g0, g1, g2, g3, spmem,
                   gsem0, gsem1, gsem2, gsem3,
                   xsem0, xsem1, xsem2, xsem3,
                   hsem0, hsem1, hsem2, hsem3):
        gbuf = (g0, g1, g2, g3)
        gsem = (gsem0, gsem1, gsem2, gsem3)
        xsem = (xsem0, xsem1, xsem2, xsem3)
        hsem = (hsem0, hsem1, hsem2, hsem3)
        cid = lax.axis_index("c")
        sid = lax.axis_index("s")
        wid = sid * NUM_CORES + cid
        base = wid * b_per_w
        pltpu.sync_copy(idx_hbm.at[wid], idx_v)

        def fire_gather(b, c):
            pltpu.async_copy(table_hbm.at[idx_v.at[c]], gbuf[b], gsem[b])

        def wait_gather(b):
            pltpu.make_async_copy(
                table_hbm.at[pl.ds(0, CHUNK)], gbuf[b], gsem[b]).wait()

        def fire_xbar(b, h, s_):
            pltpu.async_copy(
                gbuf[b].at[pl.ds(h * HALF, HALF)], spmem.at[sid, s_],
                xsem[s_])

        def wait_xbar(s_):
            pltpu.make_async_copy(
                gbuf[0].at[pl.ds(0, HALF)], spmem.at[sid, s_],
                xsem[s_]).wait()

        def fire_hbm(s_, k):
            pltpu.async_copy(
                spmem.at[sid, s_],
                out_hbm.at[pl.ds(base + k * HALF, HALF)], hsem[s_])

        def wait_hbm(s_):
            pltpu.make_async_copy(
                spmem.at[sid, s_],
                out_hbm.at[pl.ds(0, HALF)], hsem[s_]).wait()

        def scale(b):
            def row_body(i, carry):
                for j in range(D_MODEL // LANES):
                    sl = pl.ds(j * LANES, LANES)
                    gbuf[b][i, sl] = gbuf[b][i, sl] * SCALE
                return carry
            lax.fori_loop(0, CHUNK, row_body, 0, unroll=2)

        def chunk_step(b, c, first_chunk, skip_hwait, skip_gfire):
            sa = (2 * b) % NS
            sb = (2 * b + 1) % NS
            sp = (2 * b + NS - 1) % NS
            wait_gather(b)
            scale(b)
            if not skip_hwait:
                wait_hbm(sa)
            fire_xbar(b, 0, sa)
            if not first_chunk:
                wait_xbar(sp)
                fire_hbm(sp, 2 * c - 1)
            if not skip_gfire:
                fire_gather((b + AHEAD) % NB, c + AHEAD)
            if not skip_hwait:
                wait_hbm(sb)
            fire_xbar(b, 1, sb)
            wait_xbar(sa)
            fire_hbm(sa, 2 * c)

        for b in range(AHEAD):
            fire_gather(b, b)
        for b in range(NB):
            chunk_step(b, b, b == 0, b < NS // 2, False)

        def group(gi, carry):
            for b in range(NB):
                chunk_step(b, gi * NB + b, False, False, False)
            return carry

        lax.fori_loop(1, n_groups - 1, group, 0)
        for b in range(NB):
            c = n_chunks - NB + b
            chunk_step(b, c, False, False, c + AHEAD >= n_chunks)
        last = (2 * n_chunks - 1) % NS
        wait_xbar(last)
        fire_hbm(last, 2 * n_chunks - 1)
        for s_ in range(NS):
            wait_hbm(s_)

    return emb_kernel


def kernel(x, table):
    batch = x.shape[0] * x.shape[1]
    idx = x.reshape(NUM_WORKERS, batch // (NUM_WORKERS * CHUNK), CHUNK)
    idx = idx.astype(jnp.int32)
    out = _make_kernel(batch)(idx, table)
    return out.reshape(x.shape[0], x.shape[1], D_MODEL)
